# Initial kernel scaffold; baseline (speedup 1.0000x reference)
import jax
import jax.numpy as jnp
from jax.experimental import pallas as pl


def kernel(x, edge_index, W1, al1, ar1, b1, W2, al2, ar2, b2, fc1_w, fc1_b, fc2_w, fc2_b):
    raise NotImplementedError("write your pallas kernel here")


# speedup vs baseline: 3.8620x; 3.8620x over previous
import functools

import jax
import jax.numpy as jnp
from jax import lax
from jax.experimental import pallas as pl
from jax.experimental.pallas import tpu as pltpu
from jax.experimental.pallas import tpu_sc as plsc

_N = 10000
_E = 160000
_D = 256
_NC = 2
_NS = 16
_L = 16
_F = _D // 4
_ACC = 10240
_RPT = _ACC // _NS
_FPT = _RPT * _F
_K = 80
_EPT = _E // _NS
_NCH = _EPT // _K
_BLK = 1000
_NBLK = _N // _BLK

def _iota16():
    return lax.iota(jnp.int32, _L)


def _splat(v16, j):
    idx = (_iota16() * 0 + j).reshape(_L, 1)
    return lax.gather(
        v16, idx,
        lax.GatherDimensionNumbers(offset_dims=(), collapsed_slice_dims=(0,),
                                   start_index_map=(0,)),
        (1,), mode=lax.GatherScatterMode.PROMISE_IN_BOUNDS)


def _sc_edge_body(src_hbm, dst_hbm, el_hbm, er_hbm, g_hbm, h_hbm,
                  num_out, s_out,
                  srcv, dstv, giv, av, bv, exv, rows, vals, offs, gv, sv, sem,
                  acc, sacc):
    cid = lax.axis_index("c")
    sid = lax.axis_index("s")

    def zb(i, cc):
        sv[pl.ds(i * _L, _L)] = lax.convert_element_type(_iota16() * 0,
                                                         jnp.float32)
        return cc

    lax.fori_loop(0, _FPT // _L, zb, 0)
    pltpu.sync_copy(sv, acc.at[pl.ds(sid * _FPT, _FPT)])
    pltpu.sync_copy(sv.at[pl.ds(0, _RPT)], sacc.at[pl.ds(sid * _RPT, _RPT)])
    pltpu.sync_copy(g_hbm, gv)
    plsc.subcore_barrier()

    tile_e0 = sid * _EPT

    def chunk_body(c, carry):
        e0 = tile_e0 + c * _K
        pltpu.sync_copy(src_hbm.at[pl.ds(e0, _K)], srcv)
        pltpu.sync_copy(dst_hbm.at[pl.ds(e0, _K)], dstv)
        cp_a = pltpu.async_copy(el_hbm.at[srcv], av, sem)
        cp_b = pltpu.async_copy(er_hbm.at[dstv], bv, sem)
        cp_a.wait()
        cp_b.wait()

        def grp(gi, cc):
            s16 = pl.ds(gi * _L, _L)
            a = av[s16]
            b = bv[s16]
            e = a + b
            e = jnp.maximum(e, 0.2 * e)
            m = gv[...] + b
            m = jnp.maximum(m, 0.2 * m)
            exv[s16] = jnp.exp(e - m)
            giv[s16] = srcv[s16] + cid * _N
            return cc

        lax.fori_loop(0, _K // _L, grp, 0)
        cp_r = pltpu.async_copy(h_hbm.at[giv], rows, sem)
        cp_r.wait()

        def scale(gi, cc):
            base16 = pl.ds(gi * _L, _L)
            ex16 = exv[base16]
            d16 = dstv[base16]
            for j in range(_L):
                w = _splat(ex16, j)
                dof = _splat(d16, j) * _F
                k = gi * _L + j
                for q in range(_F // _L):
                    kf = pl.ds(k * _F + q * _L, _L)
                    vals[kf] = rows[k, pl.ds(q * _L, _L)] * w
                    offs[kf] = dof + (_iota16() + q * _L)
            return cc

        lax.fori_loop(0, _K // _L, scale, 0)
        pltpu.sync_copy(vals, acc.at[offs], add=True)
        pltpu.sync_copy(exv, sacc.at[dstv], add=True)
        return carry

    lax.fori_loop(0, _NCH, chunk_body, 0)
    plsc.subcore_barrier()

    pltpu.sync_copy(acc.at[pl.ds(sid * _FPT, _FPT)], sv)
    pltpu.sync_copy(sv, num_out.at[pl.ds(cid * _ACC * _F + sid * _FPT, _FPT)])
    pltpu.sync_copy(sacc.at[pl.ds(sid * _RPT, _RPT)], sv.at[pl.ds(0, _RPT)])
    pltpu.sync_copy(sv.at[pl.ds(0, _RPT)],
                    s_out.at[pl.ds(cid * _ACC + sid * _RPT, _RPT)])


@functools.cache
def _get_sc_edge():
    return functools.partial(
        pl.kernel,
        out_type=(
            jax.ShapeDtypeStruct((_NC * _ACC * _F,), jnp.float32),
            jax.ShapeDtypeStruct((_NC * _ACC,), jnp.float32),
        ),
        mesh=plsc.VectorSubcoreMesh(core_axis_name="c", subcore_axis_name="s",
                                    num_cores=_NC, num_subcores=_NS),
        compiler_params=pltpu.CompilerParams(use_tc_tiling_on_sc=False),
        scratch_types=[
            pltpu.VMEM((_K,), jnp.int32),
            pltpu.VMEM((_K,), jnp.int32),
            pltpu.VMEM((_K,), jnp.int32),
            pltpu.VMEM((_K,), jnp.float32),
            pltpu.VMEM((_K,), jnp.float32),
            pltpu.VMEM((_K,), jnp.float32),
            pltpu.VMEM((_K, _F), jnp.float32),
            pltpu.VMEM((_K * _F,), jnp.float32),
            pltpu.VMEM((_K * _F,), jnp.int32),
            pltpu.VMEM((_L,), jnp.float32),
            pltpu.VMEM((_FPT,), jnp.float32),
            pltpu.SemaphoreType.DMA,
            pltpu.VMEM_SHARED((_ACC * _F,), jnp.float32),
            pltpu.VMEM_SHARED((_ACC,), jnp.float32),
        ],
    )(_sc_edge_body)


_SELU_L = 1.0507009873554805
_SELU_A = 1.6732632423543772


def _selu(x):
    return _SELU_L * jnp.where(x > 0, x, _SELU_A * (jnp.exp(jnp.minimum(x, 0.0)) - 1.0))


def _mm_att_body(x_ref, w_ref, al_ref, ar_ref, h_ref, el_ref, er_ref, g_ref, mx_ref):
    i = pl.program_id(0)
    h = jnp.dot(x_ref[...], w_ref[...], preferred_element_type=jnp.float32,
                precision=lax.Precision.HIGHEST)
    h_ref[...] = h
    el = jnp.sum(h * al_ref[...], axis=1, keepdims=True)
    er = jnp.sum(h * ar_ref[...], axis=1, keepdims=True)
    el_ref[...] = el
    er_ref[...] = er
    bm = jnp.max(el)

    @pl.when(i == 0)
    def _():
        mx_ref[0] = bm

    @pl.when(i > 0)
    def _():
        mx_ref[0] = jnp.maximum(mx_ref[0], bm)

    @pl.when(i == pl.num_programs(0) - 1)
    def _():
        g_ref[...] = jnp.full((1, 16), mx_ref[0], jnp.float32)


def _norm_mm_att_body(num_ref, s_ref, b_ref, w_ref, al_ref, ar_ref,
                      h_ref, el_ref, er_ref, g_ref, mx_ref):
    i = pl.program_id(0)
    xin = _selu(num_ref[...] / (s_ref[...] + 1e-16) + b_ref[...])
    h = jnp.dot(xin, w_ref[...], preferred_element_type=jnp.float32,
                precision=lax.Precision.HIGHEST)
    h_ref[...] = h
    el = jnp.sum(h * al_ref[...], axis=1, keepdims=True)
    er = jnp.sum(h * ar_ref[...], axis=1, keepdims=True)
    el_ref[...] = el
    er_ref[...] = er
    bm = jnp.max(el)

    @pl.when(i == 0)
    def _():
        mx_ref[0] = bm

    @pl.when(i > 0)
    def _():
        mx_ref[0] = jnp.maximum(mx_ref[0], bm)

    @pl.when(i == pl.num_programs(0) - 1)
    def _():
        g_ref[...] = jnp.full((1, 16), mx_ref[0], jnp.float32)


def _head_body(num_ref, s_ref, b_ref, f1w_ref, f1b_ref, f2w_ref, f2b_ref,
               out_ref, acc_ref):
    i = pl.program_id(0)
    xin = _selu(num_ref[...] / (s_ref[...] + 1e-16) + b_ref[...])
    part = jnp.sum(xin, axis=0, keepdims=True)

    @pl.when(i == 0)
    def _():
        acc_ref[...] = part

    @pl.when(i > 0)
    def _():
        acc_ref[...] = acc_ref[...] + part

    @pl.when(i == pl.num_programs(0) - 1)
    def _():
        gbar = acc_ref[...] / _N
        z1 = _selu(jnp.dot(gbar, f1w_ref[...], preferred_element_type=jnp.float32,
                           precision=lax.Precision.HIGHEST) + f1b_ref[...])
        z2 = jnp.dot(z1, f2w_ref[...], preferred_element_type=jnp.float32,
                     precision=lax.Precision.HIGHEST) + f2b_ref[...]
        zm = jnp.max(z2)
        out_ref[...] = (z2 - zm) - jnp.log(jnp.sum(jnp.exp(z2 - zm)))


def _row_spec(width):
    return pl.BlockSpec((_BLK, width), lambda i: (i, 0))


def _full_spec(shape):
    return pl.BlockSpec(shape, lambda i: tuple(0 for _ in shape))


_mm_att = pl.pallas_call(
    _mm_att_body,
    grid=(_NBLK,),
    in_specs=[
        _row_spec(_D),
        _full_spec((_D, _D)),
        _full_spec((1, _D)),
        _full_spec((1, _D)),
    ],
    out_specs=[
        _row_spec(_D),
        _row_spec(1),
        _row_spec(1),
        _full_spec((1, 16)),
    ],
    out_shape=[
        jax.ShapeDtypeStruct((_N, _D), jnp.float32),
        jax.ShapeDtypeStruct((_N, 1), jnp.float32),
        jax.ShapeDtypeStruct((_N, 1), jnp.float32),
        jax.ShapeDtypeStruct((1, 16), jnp.float32),
    ],
    scratch_shapes=[pltpu.SMEM((1,), jnp.float32)],
)

_norm_mm_att = pl.pallas_call(
    _norm_mm_att_body,
    grid=(_NBLK,),
    in_specs=[
        _row_spec(_D),
        _row_spec(1),
        _full_spec((1, _D)),
        _full_spec((_D, _D)),
        _full_spec((1, _D)),
        _full_spec((1, _D)),
    ],
    out_specs=[
        _row_spec(_D),
        _row_spec(1),
        _row_spec(1),
        _full_spec((1, 16)),
    ],
    out_shape=[
        jax.ShapeDtypeStruct((_N, _D), jnp.float32),
        jax.ShapeDtypeStruct((_N, 1), jnp.float32),
        jax.ShapeDtypeStruct((_N, 1), jnp.float32),
        jax.ShapeDtypeStruct((1, 16), jnp.float32),
    ],
    scratch_shapes=[pltpu.SMEM((1,), jnp.float32)],
)

_head = pl.pallas_call(
    _head_body,
    grid=(_NBLK,),
    in_specs=[
        _row_spec(_D),
        _row_spec(1),
        _full_spec((1, _D)),
        _full_spec((_D, _D // 2)),
        _full_spec((1, _D // 2)),
        _full_spec((_D // 2, 2)),
        _full_spec((1, 2)),
    ],
    out_specs=[_full_spec((1, 2))],
    out_shape=[jax.ShapeDtypeStruct((1, 2), jnp.float32)],
    scratch_shapes=[pltpu.VMEM((1, _D), jnp.float32)],
)


def _split_tables(h):
    ta = jnp.concatenate([h[:, :_F], h[:, _F:2 * _F]], axis=0)
    tb = jnp.concatenate([h[:, 2 * _F:3 * _F], h[:, 3 * _F:]], axis=0)
    return ta, tb


def _sc_layer(src, dst, el, er, g, h):
    ta, tb = _split_tables(h)
    sc_edge = _get_sc_edge()
    args = (src, dst, el.reshape(-1), er.reshape(-1), g.reshape(-1))
    na, s = sc_edge(*args, ta)
    nb, _ = sc_edge(*args, tb)
    na = na.reshape(_NC, _ACC, _F)
    nb = nb.reshape(_NC, _ACC, _F)
    numc = jnp.concatenate(
        [na[0, :_N], na[1, :_N], nb[0, :_N], nb[1, :_N]], axis=1)
    return numc, s[:_N].reshape(-1, 1)


def kernel(x, edge_index, W1, al1, ar1, b1, W2, al2, ar2, b2,
           fc1_w, fc1_b, fc2_w, fc2_b):
    src = edge_index[0]
    dst = edge_index[1]

    h1, el1, er1, g1 = _mm_att(x, W1, al1.reshape(1, -1), ar1.reshape(1, -1))
    numc1, sc1 = _sc_layer(src, dst, el1, er1, g1, h1)

    h2, el2, er2, g2 = _norm_mm_att(numc1, sc1, b1.reshape(1, -1), W2,
                                    al2.reshape(1, -1), ar2.reshape(1, -1))
    numc2, sc2 = _sc_layer(src, dst, el2, er2, g2, h2)

    (out,) = _head(numc2, sc2, b2.reshape(1, -1), fc1_w, fc1_b.reshape(1, -1),
                   fc2_w, fc2_b.reshape(1, -1))
    return out.T


# speedup vs baseline: 5.1684x; 1.3383x over previous
import functools

import jax
import jax.numpy as jnp
from jax import lax
from jax.experimental import pallas as pl
from jax.experimental.pallas import tpu as pltpu
from jax.experimental.pallas import tpu_sc as plsc

_N = 10000
_E = 160000
_D = 256
_NC = 2
_NS = 16
_L = 16
_F = _D // 4
_ACC = 10240
_RPT = _ACC // _NS
_FPT = _RPT * _F
_K = 80
_SUB = 2000
_EPT = _E // _NS
_NCH = _EPT // _K
_BLK = 1000
_NBLK = _N // _BLK

def _iota16():
    return lax.iota(jnp.int32, _L)


def _splat(v16, j):
    idx = (_iota16() * 0 + j).reshape(_L, 1)
    return lax.gather(
        v16, idx,
        lax.GatherDimensionNumbers(offset_dims=(), collapsed_slice_dims=(0,),
                                   start_index_map=(0,)),
        (1,), mode=lax.GatherScatterMode.PROMISE_IN_BOUNDS)


def _sc_edge_body(src_hbm, dst_hbm, el_hbm, er_hbm, g_hbm, h_hbm,
                  num_out, s_out,
                  srcv, dstv, giv, av, bv, rows, vals, offs, gv, sv, sem,
                  acc, sacc):
    cid = lax.axis_index("c")
    sid = lax.axis_index("s")

    def zb(i, cc):
        sv[pl.ds(i * _L, _L)] = lax.convert_element_type(_iota16() * 0,
                                                         jnp.float32)
        return cc

    lax.fori_loop(0, _FPT // _L, zb, 0)
    pltpu.sync_copy(sv, acc.at[pl.ds(sid * _FPT, _FPT)])
    pltpu.sync_copy(sv.at[pl.ds(0, _RPT)], sacc.at[pl.ds(sid * _RPT, _RPT)])
    pltpu.sync_copy(g_hbm, gv)
    plsc.subcore_barrier()

    tile_e0 = sid * _EPT

    def sub_body(sb, carry0):
        s0 = tile_e0 + sb * _SUB
        pltpu.sync_copy(src_hbm.at[pl.ds(s0, _SUB)], srcv)
        pltpu.sync_copy(dst_hbm.at[pl.ds(s0, _SUB)], dstv)
        cp_a = pltpu.async_copy(el_hbm.at[srcv], av, sem)
        cp_b = pltpu.async_copy(er_hbm.at[dstv], bv, sem)
        cp_a.wait()
        cp_b.wait()

        def grp(gi, cc):
            s16 = pl.ds(gi * _L, _L)
            a = av[s16]
            b = bv[s16]
            e = a + b
            e = jnp.maximum(e, 0.2 * e)
            m = gv[...] + b
            m = jnp.maximum(m, 0.2 * m)
            av[s16] = jnp.exp(e - m)
            giv[s16] = srcv[s16] + cid * _N
            return cc

        lax.fori_loop(0, _SUB // _L, grp, 0)
        pltpu.sync_copy(av, sacc.at[dstv], add=True)

        def chunk_body(c, carry):
            e0 = c * _K
            cp_r = pltpu.async_copy(h_hbm.at[giv.at[pl.ds(e0, _K)]], rows, sem)
            cp_r.wait()

            def scale(gi, cc):
                base16 = pl.ds(e0 + gi * _L, _L)
                ex16 = av[base16]
                d16 = dstv[base16]
                for j in range(_L):
                    w = _splat(ex16, j)
                    dof = _splat(d16, j) * _F
                    k = gi * _L + j
                    for q in range(_F // _L):
                        kf = pl.ds(k * _F + q * _L, _L)
                        vals[kf] = rows[k, pl.ds(q * _L, _L)] * w
                        offs[kf] = dof + (_iota16() + q * _L)
                return cc

            lax.fori_loop(0, _K // _L, scale, 0)
            pltpu.sync_copy(vals, acc.at[offs], add=True)
            return carry

        lax.fori_loop(0, _SUB // _K, chunk_body, 0)
        return carry0

    lax.fori_loop(0, _EPT // _SUB, sub_body, 0)
    plsc.subcore_barrier()

    pltpu.sync_copy(acc.at[pl.ds(sid * _FPT, _FPT)], sv)
    pltpu.sync_copy(sv, num_out.at[pl.ds(cid * _ACC * _F + sid * _FPT, _FPT)])
    pltpu.sync_copy(sacc.at[pl.ds(sid * _RPT, _RPT)], sv.at[pl.ds(0, _RPT)])
    pltpu.sync_copy(sv.at[pl.ds(0, _RPT)],
                    s_out.at[pl.ds(cid * _ACC + sid * _RPT, _RPT)])


@functools.cache
def _get_sc_edge():
    return functools.partial(
        pl.kernel,
        out_type=(
            jax.ShapeDtypeStruct((_NC * _ACC * _F,), jnp.float32),
            jax.ShapeDtypeStruct((_NC * _ACC,), jnp.float32),
        ),
        mesh=plsc.VectorSubcoreMesh(core_axis_name="c", subcore_axis_name="s",
                                    num_cores=_NC, num_subcores=_NS),
        compiler_params=pltpu.CompilerParams(use_tc_tiling_on_sc=False),
        scratch_types=[
            pltpu.VMEM((_SUB,), jnp.int32),
            pltpu.VMEM((_SUB,), jnp.int32),
            pltpu.VMEM((_SUB,), jnp.int32),
            pltpu.VMEM((_SUB,), jnp.float32),
            pltpu.VMEM((_SUB,), jnp.float32),
            pltpu.VMEM((_K, _F), jnp.float32),
            pltpu.VMEM((_K * _F,), jnp.float32),
            pltpu.VMEM((_K * _F,), jnp.int32),
            pltpu.VMEM((_L,), jnp.float32),
            pltpu.VMEM((_FPT,), jnp.float32),
            pltpu.SemaphoreType.DMA,
            pltpu.VMEM_SHARED((_ACC * _F,), jnp.float32),
            pltpu.VMEM_SHARED((_ACC,), jnp.float32),
        ],
    )(_sc_edge_body)


_SELU_L = 1.0507009873554805
_SELU_A = 1.6732632423543772


def _selu(x):
    return _SELU_L * jnp.where(x > 0, x, _SELU_A * (jnp.exp(jnp.minimum(x, 0.0)) - 1.0))


def _mm_att_body(x_ref, w_ref, al_ref, ar_ref, h_ref, el_ref, er_ref, g_ref, mx_ref):
    i = pl.program_id(0)
    h = jnp.dot(x_ref[...], w_ref[...], preferred_element_type=jnp.float32,
                precision=lax.Precision.HIGHEST)
    h_ref[...] = h
    el = jnp.sum(h * al_ref[...], axis=1, keepdims=True)
    er = jnp.sum(h * ar_ref[...], axis=1, keepdims=True)
    el_ref[...] = el
    er_ref[...] = er
    bm = jnp.max(el)

    @pl.when(i == 0)
    def _():
        mx_ref[0] = bm

    @pl.when(i > 0)
    def _():
        mx_ref[0] = jnp.maximum(mx_ref[0], bm)

    @pl.when(i == pl.num_programs(0) - 1)
    def _():
        g_ref[...] = jnp.full((1, 16), mx_ref[0], jnp.float32)


def _norm_mm_att_body(num_ref, s_ref, b_ref, w_ref, al_ref, ar_ref,
                      h_ref, el_ref, er_ref, g_ref, mx_ref):
    i = pl.program_id(0)
    xin = _selu(num_ref[...] / (s_ref[...] + 1e-16) + b_ref[...])
    h = jnp.dot(xin, w_ref[...], preferred_element_type=jnp.float32,
                precision=lax.Precision.HIGHEST)
    h_ref[...] = h
    el = jnp.sum(h * al_ref[...], axis=1, keepdims=True)
    er = jnp.sum(h * ar_ref[...], axis=1, keepdims=True)
    el_ref[...] = el
    er_ref[...] = er
    bm = jnp.max(el)

    @pl.when(i == 0)
    def _():
        mx_ref[0] = bm

    @pl.when(i > 0)
    def _():
        mx_ref[0] = jnp.maximum(mx_ref[0], bm)

    @pl.when(i == pl.num_programs(0) - 1)
    def _():
        g_ref[...] = jnp.full((1, 16), mx_ref[0], jnp.float32)


def _head_body(num_ref, s_ref, b_ref, f1w_ref, f1b_ref, f2w_ref, f2b_ref,
               out_ref, acc_ref):
    i = pl.program_id(0)
    xin = _selu(num_ref[...] / (s_ref[...] + 1e-16) + b_ref[...])
    part = jnp.sum(xin, axis=0, keepdims=True)

    @pl.when(i == 0)
    def _():
        acc_ref[...] = part

    @pl.when(i > 0)
    def _():
        acc_ref[...] = acc_ref[...] + part

    @pl.when(i == pl.num_programs(0) - 1)
    def _():
        gbar = acc_ref[...] / _N
        z1 = _selu(jnp.dot(gbar, f1w_ref[...], preferred_element_type=jnp.float32,
                           precision=lax.Precision.HIGHEST) + f1b_ref[...])
        z2 = jnp.dot(z1, f2w_ref[...], preferred_element_type=jnp.float32,
                     precision=lax.Precision.HIGHEST) + f2b_ref[...]
        zm = jnp.max(z2)
        out_ref[...] = (z2 - zm) - jnp.log(jnp.sum(jnp.exp(z2 - zm)))


def _row_spec(width):
    return pl.BlockSpec((_BLK, width), lambda i: (i, 0))


def _full_spec(shape):
    return pl.BlockSpec(shape, lambda i: tuple(0 for _ in shape))


_mm_att = pl.pallas_call(
    _mm_att_body,
    grid=(_NBLK,),
    in_specs=[
        _row_spec(_D),
        _full_spec((_D, _D)),
        _full_spec((1, _D)),
        _full_spec((1, _D)),
    ],
    out_specs=[
        _row_spec(_D),
        _row_spec(1),
        _row_spec(1),
        _full_spec((1, 16)),
    ],
    out_shape=[
        jax.ShapeDtypeStruct((_N, _D), jnp.float32),
        jax.ShapeDtypeStruct((_N, 1), jnp.float32),
        jax.ShapeDtypeStruct((_N, 1), jnp.float32),
        jax.ShapeDtypeStruct((1, 16), jnp.float32),
    ],
    scratch_shapes=[pltpu.SMEM((1,), jnp.float32)],
)

_norm_mm_att = pl.pallas_call(
    _norm_mm_att_body,
    grid=(_NBLK,),
    in_specs=[
        _row_spec(_D),
        _row_spec(1),
        _full_spec((1, _D)),
        _full_spec((_D, _D)),
        _full_spec((1, _D)),
        _full_spec((1, _D)),
    ],
    out_specs=[
        _row_spec(_D),
        _row_spec(1),
        _row_spec(1),
        _full_spec((1, 16)),
    ],
    out_shape=[
        jax.ShapeDtypeStruct((_N, _D), jnp.float32),
        jax.ShapeDtypeStruct((_N, 1), jnp.float32),
        jax.ShapeDtypeStruct((_N, 1), jnp.float32),
        jax.ShapeDtypeStruct((1, 16), jnp.float32),
    ],
    scratch_shapes=[pltpu.SMEM((1,), jnp.float32)],
)

_head = pl.pallas_call(
    _head_body,
    grid=(_NBLK,),
    in_specs=[
        _row_spec(_D),
        _row_spec(1),
        _full_spec((1, _D)),
        _full_spec((_D, _D // 2)),
        _full_spec((1, _D // 2)),
        _full_spec((_D // 2, 2)),
        _full_spec((1, 2)),
    ],
    out_specs=[_full_spec((1, 2))],
    out_shape=[jax.ShapeDtypeStruct((1, 2), jnp.float32)],
    scratch_shapes=[pltpu.VMEM((1, _D), jnp.float32)],
)


def _split_tables(h):
    ta = jnp.concatenate([h[:, :_F], h[:, _F:2 * _F]], axis=0)
    tb = jnp.concatenate([h[:, 2 * _F:3 * _F], h[:, 3 * _F:]], axis=0)
    return ta, tb


def _sc_layer(src, dst, el, er, g, h):
    ta, tb = _split_tables(h)
    sc_edge = _get_sc_edge()
    args = (src, dst, el.reshape(-1), er.reshape(-1), g.reshape(-1))
    na, s = sc_edge(*args, ta)
    nb, _ = sc_edge(*args, tb)
    na = na.reshape(_NC, _ACC, _F)
    nb = nb.reshape(_NC, _ACC, _F)
    numc = jnp.concatenate(
        [na[0, :_N], na[1, :_N], nb[0, :_N], nb[1, :_N]], axis=1)
    return numc, s[:_N].reshape(-1, 1)


def kernel(x, edge_index, W1, al1, ar1, b1, W2, al2, ar2, b2,
           fc1_w, fc1_b, fc2_w, fc2_b):
    src = edge_index[0]
    dst = edge_index[1]

    h1, el1, er1, g1 = _mm_att(x, W1, al1.reshape(1, -1), ar1.reshape(1, -1))
    numc1, sc1 = _sc_layer(src, dst, el1, er1, g1, h1)

    h2, el2, er2, g2 = _norm_mm_att(numc1, sc1, b1.reshape(1, -1), W2,
                                    al2.reshape(1, -1), ar2.reshape(1, -1))
    numc2, sc2 = _sc_layer(src, dst, el2, er2, g2, h2)

    (out,) = _head(numc2, sc2, b2.reshape(1, -1), fc1_w, fc1_b.reshape(1, -1),
                   fc2_w, fc2_b.reshape(1, -1))
    return out.T


# speedup vs baseline: 6.9963x; 1.3537x over previous
import functools

import jax
import jax.numpy as jnp
from jax import lax
from jax.experimental import pallas as pl
from jax.experimental.pallas import tpu as pltpu
from jax.experimental.pallas import tpu_sc as plsc

_N = 10000
_E = 160000
_D = 256
_NC = 2
_NS = 16
_L = 16
_F = _D // 4
_ACC = 10240
_RPT = _ACC // _NS
_FPT = _RPT * _F
_SV = 8192
_K = 80
_SUB = 2000
_EPT = _E // _NS
_NCH = _EPT // _K
_BLK = 1000
_NBLK = _N // _BLK

def _iota16():
    return lax.iota(jnp.int32, _L)


def _splat(v16, j):
    idx = (_iota16() * 0 + j).reshape(_L, 1)
    return lax.gather(
        v16, idx,
        lax.GatherDimensionNumbers(offset_dims=(), collapsed_slice_dims=(0,),
                                   start_index_map=(0,)),
        (1,), mode=lax.GatherScatterMode.PROMISE_IN_BOUNDS)


def _sc_edge_body(src_hbm, dst_hbm, el_hbm, er_hbm, g_hbm, h_hbm,
                  num_out, s_out,
                  srcv, dstv, giv, av, bv, rows, vals, offs, gv, sv, sem,
                  semr, sems, acc, sacc):
    cid = lax.axis_index("c")
    sid = lax.axis_index("s")

    def zb(i, cc):
        sv[pl.ds(i * _L, _L)] = lax.convert_element_type(_iota16() * 0,
                                                         jnp.float32)
        return cc

    lax.fori_loop(0, _SV // _L, zb, 0)

    def zcp(i, cc):
        pltpu.sync_copy(sv, acc.at[pl.ds(sid * _FPT + i * _SV, _SV)])
        return cc

    lax.fori_loop(0, _FPT // _SV, zcp, 0)
    pltpu.sync_copy(sv.at[pl.ds(0, _RPT)], sacc.at[pl.ds(sid * _RPT, _RPT)])
    pltpu.sync_copy(g_hbm, gv)
    plsc.subcore_barrier()

    tile_e0 = sid * _EPT

    def sub_body(sb, carry0):
        s0 = tile_e0 + sb * _SUB
        pltpu.sync_copy(src_hbm.at[pl.ds(s0, _SUB)], srcv)
        pltpu.sync_copy(dst_hbm.at[pl.ds(s0, _SUB)], dstv)
        cp_a = pltpu.async_copy(el_hbm.at[srcv], av, sem)
        cp_b = pltpu.async_copy(er_hbm.at[dstv], bv, sem)
        cp_a.wait()
        cp_b.wait()

        def grp(gi, cc):
            s16 = pl.ds(gi * _L, _L)
            a = av[s16]
            b = bv[s16]
            e = a + b
            e = jnp.maximum(e, 0.2 * e)
            m = gv[...] + b
            m = jnp.maximum(m, 0.2 * m)
            av[s16] = jnp.exp(e - m)
            giv[s16] = srcv[s16] + cid * _N
            return cc

        lax.fori_loop(0, _SUB // _L, grp, 0)
        pltpu.sync_copy(av, sacc.at[dstv], add=True)

        kfsz = _K * _F
        nchs = _SUB // _K

        def rows_cp(c, b):
            return pltpu.make_async_copy(
                h_hbm.at[giv.at[pl.ds(c * _K, _K)]],
                rows.at[pl.ds(b * _K, _K)], semr)

        def scat_cp(b):
            return pltpu.make_async_copy(
                vals.at[pl.ds(b * kfsz, kfsz)],
                acc.at[offs.at[pl.ds(b * kfsz, kfsz)]], sems)

        pltpu.async_copy(h_hbm.at[giv.at[pl.ds(0, _K)]],
                         rows.at[pl.ds(0, _K)], semr)

        def chunk_body(c, carry):
            b = lax.rem(c, 2)
            rows_cp(c, b).wait()

            @pl.when(c + 1 < nchs)
            def _():
                pltpu.async_copy(
                    h_hbm.at[giv.at[pl.ds((c + 1) * _K, _K)]],
                    rows.at[pl.ds((1 - b) * _K, _K)], semr)

            @pl.when(c >= 2)
            def _():
                scat_cp(b).wait()

            def scale(gi, cc):
                base16 = pl.ds(c * _K + gi * _L, _L)
                ex16 = av[base16]
                d16 = dstv[base16]
                for j in range(_L):
                    w = _splat(ex16, j)
                    dof = _splat(d16, j) * _F
                    k = gi * _L + j
                    for q in range(_F // _L):
                        kf = pl.ds(b * kfsz + k * _F + q * _L, _L)
                        vals[kf] = rows[b * _K + k, pl.ds(q * _L, _L)] * w
                        offs[kf] = dof + (_iota16() + q * _L)
                return cc

            lax.fori_loop(0, _K // _L, scale, 0)
            pltpu.async_copy(vals.at[pl.ds(b * kfsz, kfsz)],
                             acc.at[offs.at[pl.ds(b * kfsz, kfsz)]], sems,
                             add=True)
            return carry

        lax.fori_loop(0, nchs, chunk_body, 0)
        scat_cp((nchs - 2) % 2).wait()
        scat_cp((nchs - 1) % 2).wait()
        return carry0

    lax.fori_loop(0, _EPT // _SUB, sub_body, 0)
    plsc.subcore_barrier()

    def ocp(i, cc):
        pltpu.sync_copy(acc.at[pl.ds(sid * _FPT + i * _SV, _SV)], sv)
        pltpu.sync_copy(sv, num_out.at[
            pl.ds(cid * _ACC * _F + sid * _FPT + i * _SV, _SV)])
        return cc

    lax.fori_loop(0, _FPT // _SV, ocp, 0)
    pltpu.sync_copy(sacc.at[pl.ds(sid * _RPT, _RPT)], sv.at[pl.ds(0, _RPT)])
    pltpu.sync_copy(sv.at[pl.ds(0, _RPT)],
                    s_out.at[pl.ds(cid * _ACC + sid * _RPT, _RPT)])


@functools.cache
def _get_sc_edge():
    return functools.partial(
        pl.kernel,
        out_type=(
            jax.ShapeDtypeStruct((_NC * _ACC * _F,), jnp.float32),
            jax.ShapeDtypeStruct((_NC * _ACC,), jnp.float32),
        ),
        mesh=plsc.VectorSubcoreMesh(core_axis_name="c", subcore_axis_name="s",
                                    num_cores=_NC, num_subcores=_NS),
        compiler_params=pltpu.CompilerParams(use_tc_tiling_on_sc=False),
        scratch_types=[
            pltpu.VMEM((_SUB,), jnp.int32),
            pltpu.VMEM((_SUB,), jnp.int32),
            pltpu.VMEM((_SUB,), jnp.int32),
            pltpu.VMEM((_SUB,), jnp.float32),
            pltpu.VMEM((_SUB,), jnp.float32),
            pltpu.VMEM((2 * _K, _F), jnp.float32),
            pltpu.VMEM((2 * _K * _F,), jnp.float32),
            pltpu.VMEM((2 * _K * _F,), jnp.int32),
            pltpu.VMEM((_L,), jnp.float32),
            pltpu.VMEM((_SV,), jnp.float32),
            pltpu.SemaphoreType.DMA,
            pltpu.SemaphoreType.DMA,
            pltpu.SemaphoreType.DMA,
            pltpu.VMEM_SHARED((_ACC * _F,), jnp.float32),
            pltpu.VMEM_SHARED((_ACC,), jnp.float32),
        ],
    )(_sc_edge_body)


_SELU_L = 1.0507009873554805
_SELU_A = 1.6732632423543772


def _selu(x):
    return _SELU_L * jnp.where(x > 0, x, _SELU_A * (jnp.exp(jnp.minimum(x, 0.0)) - 1.0))


def _mm_att_body(x_ref, w_ref, al_ref, ar_ref, h_ref, el_ref, er_ref, g_ref, mx_ref):
    i = pl.program_id(0)
    h = jnp.dot(x_ref[...], w_ref[...], preferred_element_type=jnp.float32,
                precision=lax.Precision.HIGHEST)
    h_ref[...] = h
    el = jnp.sum(h * al_ref[...], axis=1, keepdims=True)
    er = jnp.sum(h * ar_ref[...], axis=1, keepdims=True)
    el_ref[...] = el
    er_ref[...] = er
    bm = jnp.max(el)

    @pl.when(i == 0)
    def _():
        mx_ref[0] = bm

    @pl.when(i > 0)
    def _():
        mx_ref[0] = jnp.maximum(mx_ref[0], bm)

    @pl.when(i == pl.num_programs(0) - 1)
    def _():
        g_ref[...] = jnp.full((1, 16), mx_ref[0], jnp.float32)


def _norm_mm_att_body(num_ref, s_ref, b_ref, w_ref, al_ref, ar_ref,
                      h_ref, el_ref, er_ref, g_ref, mx_ref):
    i = pl.program_id(0)
    xin = _selu(num_ref[...] / (s_ref[...] + 1e-16) + b_ref[...])
    h = jnp.dot(xin, w_ref[...], preferred_element_type=jnp.float32,
                precision=lax.Precision.HIGHEST)
    h_ref[...] = h
    el = jnp.sum(h * al_ref[...], axis=1, keepdims=True)
    er = jnp.sum(h * ar_ref[...], axis=1, keepdims=True)
    el_ref[...] = el
    er_ref[...] = er
    bm = jnp.max(el)

    @pl.when(i == 0)
    def _():
        mx_ref[0] = bm

    @pl.when(i > 0)
    def _():
        mx_ref[0] = jnp.maximum(mx_ref[0], bm)

    @pl.when(i == pl.num_programs(0) - 1)
    def _():
        g_ref[...] = jnp.full((1, 16), mx_ref[0], jnp.float32)


def _head_body(num_ref, s_ref, b_ref, f1w_ref, f1b_ref, f2w_ref, f2b_ref,
               out_ref, acc_ref):
    i = pl.program_id(0)
    xin = _selu(num_ref[...] / (s_ref[...] + 1e-16) + b_ref[...])
    part = jnp.sum(xin, axis=0, keepdims=True)

    @pl.when(i == 0)
    def _():
        acc_ref[...] = part

    @pl.when(i > 0)
    def _():
        acc_ref[...] = acc_ref[...] + part

    @pl.when(i == pl.num_programs(0) - 1)
    def _():
        gbar = acc_ref[...] / _N
        z1 = _selu(jnp.dot(gbar, f1w_ref[...], preferred_element_type=jnp.float32,
                           precision=lax.Precision.HIGHEST) + f1b_ref[...])
        z2 = jnp.dot(z1, f2w_ref[...], preferred_element_type=jnp.float32,
                     precision=lax.Precision.HIGHEST) + f2b_ref[...]
        zm = jnp.max(z2)
        out_ref[...] = (z2 - zm) - jnp.log(jnp.sum(jnp.exp(z2 - zm)))


def _row_spec(width):
    return pl.BlockSpec((_BLK, width), lambda i: (i, 0))


def _full_spec(shape):
    return pl.BlockSpec(shape, lambda i: tuple(0 for _ in shape))


_mm_att = pl.pallas_call(
    _mm_att_body,
    grid=(_NBLK,),
    in_specs=[
        _row_spec(_D),
        _full_spec((_D, _D)),
        _full_spec((1, _D)),
        _full_spec((1, _D)),
    ],
    out_specs=[
        _row_spec(_D),
        _row_spec(1),
        _row_spec(1),
        _full_spec((1, 16)),
    ],
    out_shape=[
        jax.ShapeDtypeStruct((_N, _D), jnp.float32),
        jax.ShapeDtypeStruct((_N, 1), jnp.float32),
        jax.ShapeDtypeStruct((_N, 1), jnp.float32),
        jax.ShapeDtypeStruct((1, 16), jnp.float32),
    ],
    scratch_shapes=[pltpu.SMEM((1,), jnp.float32)],
)

_norm_mm_att = pl.pallas_call(
    _norm_mm_att_body,
    grid=(_NBLK,),
    in_specs=[
        _row_spec(_D),
        _row_spec(1),
        _full_spec((1, _D)),
        _full_spec((_D, _D)),
        _full_spec((1, _D)),
        _full_spec((1, _D)),
    ],
    out_specs=[
        _row_spec(_D),
        _row_spec(1),
        _row_spec(1),
        _full_spec((1, 16)),
    ],
    out_shape=[
        jax.ShapeDtypeStruct((_N, _D), jnp.float32),
        jax.ShapeDtypeStruct((_N, 1), jnp.float32),
        jax.ShapeDtypeStruct((_N, 1), jnp.float32),
        jax.ShapeDtypeStruct((1, 16), jnp.float32),
    ],
    scratch_shapes=[pltpu.SMEM((1,), jnp.float32)],
)

_head = pl.pallas_call(
    _head_body,
    grid=(_NBLK,),
    in_specs=[
        _row_spec(_D),
        _row_spec(1),
        _full_spec((1, _D)),
        _full_spec((_D, _D // 2)),
        _full_spec((1, _D // 2)),
        _full_spec((_D // 2, 2)),
        _full_spec((1, 2)),
    ],
    out_specs=[_full_spec((1, 2))],
    out_shape=[jax.ShapeDtypeStruct((1, 2), jnp.float32)],
    scratch_shapes=[pltpu.VMEM((1, _D), jnp.float32)],
)


def _split_tables(h):
    ta = jnp.concatenate([h[:, :_F], h[:, _F:2 * _F]], axis=0)
    tb = jnp.concatenate([h[:, 2 * _F:3 * _F], h[:, 3 * _F:]], axis=0)
    return ta, tb


def _sc_layer(src, dst, el, er, g, h):
    ta, tb = _split_tables(h)
    sc_edge = _get_sc_edge()
    args = (src, dst, el.reshape(-1), er.reshape(-1), g.reshape(-1))
    na, s = sc_edge(*args, ta)
    nb, _ = sc_edge(*args, tb)
    na = na.reshape(_NC, _ACC, _F)
    nb = nb.reshape(_NC, _ACC, _F)
    numc = jnp.concatenate(
        [na[0, :_N], na[1, :_N], nb[0, :_N], nb[1, :_N]], axis=1)
    return numc, s[:_N].reshape(-1, 1)


def kernel(x, edge_index, W1, al1, ar1, b1, W2, al2, ar2, b2,
           fc1_w, fc1_b, fc2_w, fc2_b):
    src = edge_index[0]
    dst = edge_index[1]

    h1, el1, er1, g1 = _mm_att(x, W1, al1.reshape(1, -1), ar1.reshape(1, -1))
    numc1, sc1 = _sc_layer(src, dst, el1, er1, g1, h1)

    h2, el2, er2, g2 = _norm_mm_att(numc1, sc1, b1.reshape(1, -1), W2,
                                    al2.reshape(1, -1), ar2.reshape(1, -1))
    numc2, sc2 = _sc_layer(src, dst, el2, er2, g2, h2)

    (out,) = _head(numc2, sc2, b2.reshape(1, -1), fc1_w, fc1_b.reshape(1, -1),
                   fc2_w, fc2_b.reshape(1, -1))
    return out.T


# speedup vs baseline: 7.0027x; 1.0009x over previous
import functools

import jax
import jax.numpy as jnp
from jax import lax
from jax.experimental import pallas as pl
from jax.experimental.pallas import tpu as pltpu
from jax.experimental.pallas import tpu_sc as plsc

_N = 10000
_E = 160000
_D = 256
_NC = 2
_NS = 16
_L = 16
_F = _D // 4
_ACC = 10240
_RPT = _ACC // _NS
_FPT = _RPT * _F
_SV = 8192
_K = 80
_SUB = 2000
_EPT = _E // _NS
_NCH = _EPT // _K
_BLK = 1000
_NBLK = _N // _BLK

def _iota16():
    return lax.iota(jnp.int32, _L)


def _splat(v16, j):
    idx = (_iota16() * 0 + j).reshape(_L, 1)
    return lax.gather(
        v16, idx,
        lax.GatherDimensionNumbers(offset_dims=(), collapsed_slice_dims=(0,),
                                   start_index_map=(0,)),
        (1,), mode=lax.GatherScatterMode.PROMISE_IN_BOUNDS)


def _sc_edge_body(src_hbm, dst_hbm, el_hbm, er_hbm, g_hbm, h_hbm,
                  num_out, s_out,
                  srcv, dstv, giv, av, bv, rows, vals, offs, gv, sv, sem,
                  semr, sems, acc, sacc):
    cid = lax.axis_index("c")
    sid = lax.axis_index("s")

    def zb(i, cc):
        sv[pl.ds(i * _L, _L)] = lax.convert_element_type(_iota16() * 0,
                                                         jnp.float32)
        return cc

    lax.fori_loop(0, _SV // _L, zb, 0)

    def zcp(i, cc):
        pltpu.sync_copy(sv, acc.at[pl.ds(sid * _FPT + i * _SV, _SV)])
        return cc

    lax.fori_loop(0, _FPT // _SV, zcp, 0)
    pltpu.sync_copy(sv.at[pl.ds(0, _RPT)], sacc.at[pl.ds(sid * _RPT, _RPT)])
    pltpu.sync_copy(g_hbm, gv)
    plsc.subcore_barrier()

    tile_e0 = sid * _EPT

    def sub_body(sb, carry0):
        s0 = tile_e0 + sb * _SUB
        pltpu.sync_copy(src_hbm.at[pl.ds(s0, _SUB)], srcv)
        pltpu.sync_copy(dst_hbm.at[pl.ds(s0, _SUB)], dstv)
        cp_a = pltpu.async_copy(el_hbm.at[srcv], av, sem)
        cp_b = pltpu.async_copy(er_hbm.at[dstv], bv, sem)
        cp_a.wait()
        cp_b.wait()

        def grp(gi, cc):
            s16 = pl.ds(gi * _L, _L)
            a = av[s16]
            b = bv[s16]
            e = a + b
            e = jnp.maximum(e, 0.2 * e)
            m = gv[...] + b
            m = jnp.maximum(m, 0.2 * m)
            av[s16] = jnp.exp(e - m)
            giv[s16] = srcv[s16] + cid * _N
            return cc

        lax.fori_loop(0, _SUB // _L, grp, 0)
        pltpu.sync_copy(av, sacc.at[dstv], add=True)

        kfsz = _K * _F
        nchs = _SUB // _K

        def rows_cp(c, b):
            return pltpu.make_async_copy(
                h_hbm.at[giv.at[pl.ds(c * _K, _K)]],
                rows.at[pl.ds(b * _K, _K)], semr)

        def scat_cp(b):
            return pltpu.make_async_copy(
                vals.at[pl.ds(b * kfsz, kfsz)],
                acc.at[offs.at[pl.ds(b * kfsz, kfsz)]], sems)

        pltpu.async_copy(h_hbm.at[giv.at[pl.ds(0, _K)]],
                         rows.at[pl.ds(0, _K)], semr)

        def chunk_body(c, carry):
            b = lax.rem(c, 2)
            rows_cp(c, b).wait()

            @pl.when(c + 1 < nchs)
            def _():
                pltpu.async_copy(
                    h_hbm.at[giv.at[pl.ds((c + 1) * _K, _K)]],
                    rows.at[pl.ds((1 - b) * _K, _K)], semr)

            @pl.when(c >= 2)
            def _():
                scat_cp(b).wait()

            def scale(gi, cc):
                base16 = pl.ds(c * _K + gi * _L, _L)
                ex16 = av[base16]
                dof16 = dstv[base16] * _F
                for j in range(_L):
                    w = _splat(ex16, j)
                    dof = _splat(dof16, j)
                    k = gi * _L + j
                    for q in range(_F // _L):
                        kf = pl.ds(b * kfsz + k * _F + q * _L, _L)
                        vals[kf] = rows[b * _K + k, pl.ds(q * _L, _L)] * w
                        offs[kf] = dof + (_iota16() + q * _L)
                return cc

            lax.fori_loop(0, _K // _L, scale, 0)
            pltpu.async_copy(vals.at[pl.ds(b * kfsz, kfsz)],
                             acc.at[offs.at[pl.ds(b * kfsz, kfsz)]], sems,
                             add=True)
            return carry

        lax.fori_loop(0, nchs, chunk_body, 0)
        scat_cp((nchs - 2) % 2).wait()
        scat_cp((nchs - 1) % 2).wait()
        return carry0

    lax.fori_loop(0, _EPT // _SUB, sub_body, 0)
    plsc.subcore_barrier()

    def ocp(i, cc):
        pltpu.sync_copy(acc.at[pl.ds(sid * _FPT + i * _SV, _SV)], sv)
        pltpu.sync_copy(sv, num_out.at[
            pl.ds(cid * _ACC * _F + sid * _FPT + i * _SV, _SV)])
        return cc

    lax.fori_loop(0, _FPT // _SV, ocp, 0)
    pltpu.sync_copy(sacc.at[pl.ds(sid * _RPT, _RPT)], sv.at[pl.ds(0, _RPT)])
    pltpu.sync_copy(sv.at[pl.ds(0, _RPT)],
                    s_out.at[pl.ds(cid * _ACC + sid * _RPT, _RPT)])


@functools.cache
def _get_sc_edge():
    return functools.partial(
        pl.kernel,
        out_type=(
            jax.ShapeDtypeStruct((_NC * _ACC * _F,), jnp.float32),
            jax.ShapeDtypeStruct((_NC * _ACC,), jnp.float32),
        ),
        mesh=plsc.VectorSubcoreMesh(core_axis_name="c", subcore_axis_name="s",
                                    num_cores=_NC, num_subcores=_NS),
        compiler_params=pltpu.CompilerParams(use_tc_tiling_on_sc=False),
        scratch_types=[
            pltpu.VMEM((_SUB,), jnp.int32),
            pltpu.VMEM((_SUB,), jnp.int32),
            pltpu.VMEM((_SUB,), jnp.int32),
            pltpu.VMEM((_SUB,), jnp.float32),
            pltpu.VMEM((_SUB,), jnp.float32),
            pltpu.VMEM((2 * _K, _F), jnp.float32),
            pltpu.VMEM((2 * _K * _F,), jnp.float32),
            pltpu.VMEM((2 * _K * _F,), jnp.int32),
            pltpu.VMEM((_L,), jnp.float32),
            pltpu.VMEM((_SV,), jnp.float32),
            pltpu.SemaphoreType.DMA,
            pltpu.SemaphoreType.DMA,
            pltpu.SemaphoreType.DMA,
            pltpu.VMEM_SHARED((_ACC * _F,), jnp.float32),
            pltpu.VMEM_SHARED((_ACC,), jnp.float32),
        ],
    )(_sc_edge_body)


_SELU_L = 1.0507009873554805
_SELU_A = 1.6732632423543772


def _selu(x):
    return _SELU_L * jnp.where(x > 0, x, _SELU_A * (jnp.exp(jnp.minimum(x, 0.0)) - 1.0))


def _mm_att_body(x_ref, w_ref, al_ref, ar_ref, h_ref, el_ref, er_ref, g_ref, mx_ref):
    i = pl.program_id(0)
    h = jnp.dot(x_ref[...], w_ref[...], preferred_element_type=jnp.float32,
                precision=lax.Precision.HIGHEST)
    h_ref[...] = h
    el = jnp.sum(h * al_ref[...], axis=1, keepdims=True)
    er = jnp.sum(h * ar_ref[...], axis=1, keepdims=True)
    el_ref[...] = el
    er_ref[...] = er
    bm = jnp.max(el)

    @pl.when(i == 0)
    def _():
        mx_ref[0] = bm

    @pl.when(i > 0)
    def _():
        mx_ref[0] = jnp.maximum(mx_ref[0], bm)

    @pl.when(i == pl.num_programs(0) - 1)
    def _():
        g_ref[...] = jnp.full((1, 16), mx_ref[0], jnp.float32)


def _norm_mm_att_body(num_ref, s_ref, b_ref, w_ref, al_ref, ar_ref,
                      h_ref, el_ref, er_ref, g_ref, mx_ref):
    i = pl.program_id(0)
    xin = _selu(num_ref[...] / (s_ref[...] + 1e-16) + b_ref[...])
    h = jnp.dot(xin, w_ref[...], preferred_element_type=jnp.float32,
                precision=lax.Precision.HIGHEST)
    h_ref[...] = h
    el = jnp.sum(h * al_ref[...], axis=1, keepdims=True)
    er = jnp.sum(h * ar_ref[...], axis=1, keepdims=True)
    el_ref[...] = el
    er_ref[...] = er
    bm = jnp.max(el)

    @pl.when(i == 0)
    def _():
        mx_ref[0] = bm

    @pl.when(i > 0)
    def _():
        mx_ref[0] = jnp.maximum(mx_ref[0], bm)

    @pl.when(i == pl.num_programs(0) - 1)
    def _():
        g_ref[...] = jnp.full((1, 16), mx_ref[0], jnp.float32)


def _head_body(num_ref, s_ref, b_ref, f1w_ref, f1b_ref, f2w_ref, f2b_ref,
               out_ref, acc_ref):
    i = pl.program_id(0)
    xin = _selu(num_ref[...] / (s_ref[...] + 1e-16) + b_ref[...])
    part = jnp.sum(xin, axis=0, keepdims=True)

    @pl.when(i == 0)
    def _():
        acc_ref[...] = part

    @pl.when(i > 0)
    def _():
        acc_ref[...] = acc_ref[...] + part

    @pl.when(i == pl.num_programs(0) - 1)
    def _():
        gbar = acc_ref[...] / _N
        z1 = _selu(jnp.dot(gbar, f1w_ref[...], preferred_element_type=jnp.float32,
                           precision=lax.Precision.HIGHEST) + f1b_ref[...])
        z2 = jnp.dot(z1, f2w_ref[...], preferred_element_type=jnp.float32,
                     precision=lax.Precision.HIGHEST) + f2b_ref[...]
        zm = jnp.max(z2)
        out_ref[...] = (z2 - zm) - jnp.log(jnp.sum(jnp.exp(z2 - zm)))


def _row_spec(width):
    return pl.BlockSpec((_BLK, width), lambda i: (i, 0))


def _full_spec(shape):
    return pl.BlockSpec(shape, lambda i: tuple(0 for _ in shape))


_mm_att = pl.pallas_call(
    _mm_att_body,
    grid=(_NBLK,),
    in_specs=[
        _row_spec(_D),
        _full_spec((_D, _D)),
        _full_spec((1, _D)),
        _full_spec((1, _D)),
    ],
    out_specs=[
        _row_spec(_D),
        _row_spec(1),
        _row_spec(1),
        _full_spec((1, 16)),
    ],
    out_shape=[
        jax.ShapeDtypeStruct((_N, _D), jnp.float32),
        jax.ShapeDtypeStruct((_N, 1), jnp.float32),
        jax.ShapeDtypeStruct((_N, 1), jnp.float32),
        jax.ShapeDtypeStruct((1, 16), jnp.float32),
    ],
    scratch_shapes=[pltpu.SMEM((1,), jnp.float32)],
)

_norm_mm_att = pl.pallas_call(
    _norm_mm_att_body,
    grid=(_NBLK,),
    in_specs=[
        _row_spec(_D),
        _row_spec(1),
        _full_spec((1, _D)),
        _full_spec((_D, _D)),
        _full_spec((1, _D)),
        _full_spec((1, _D)),
    ],
    out_specs=[
        _row_spec(_D),
        _row_spec(1),
        _row_spec(1),
        _full_spec((1, 16)),
    ],
    out_shape=[
        jax.ShapeDtypeStruct((_N, _D), jnp.float32),
        jax.ShapeDtypeStruct((_N, 1), jnp.float32),
        jax.ShapeDtypeStruct((_N, 1), jnp.float32),
        jax.ShapeDtypeStruct((1, 16), jnp.float32),
    ],
    scratch_shapes=[pltpu.SMEM((1,), jnp.float32)],
)

_head = pl.pallas_call(
    _head_body,
    grid=(_NBLK,),
    in_specs=[
        _row_spec(_D),
        _row_spec(1),
        _full_spec((1, _D)),
        _full_spec((_D, _D // 2)),
        _full_spec((1, _D // 2)),
        _full_spec((_D // 2, 2)),
        _full_spec((1, 2)),
    ],
    out_specs=[_full_spec((1, 2))],
    out_shape=[jax.ShapeDtypeStruct((1, 2), jnp.float32)],
    scratch_shapes=[pltpu.VMEM((1, _D), jnp.float32)],
)


def _split_tables(h):
    ta = jnp.concatenate([h[:, :_F], h[:, _F:2 * _F]], axis=0)
    tb = jnp.concatenate([h[:, 2 * _F:3 * _F], h[:, 3 * _F:]], axis=0)
    return ta, tb


def _sc_layer(src, dst, el, er, g, h):
    ta, tb = _split_tables(h)
    sc_edge = _get_sc_edge()
    args = (src, dst, el.reshape(-1), er.reshape(-1), g.reshape(-1))
    na, s = sc_edge(*args, ta)
    nb, _ = sc_edge(*args, tb)
    na = na.reshape(_NC, _ACC, _F)
    nb = nb.reshape(_NC, _ACC, _F)
    numc = jnp.concatenate(
        [na[0, :_N], na[1, :_N], nb[0, :_N], nb[1, :_N]], axis=1)
    return numc, s[:_N].reshape(-1, 1)


def kernel(x, edge_index, W1, al1, ar1, b1, W2, al2, ar2, b2,
           fc1_w, fc1_b, fc2_w, fc2_b):
    src = edge_index[0]
    dst = edge_index[1]

    h1, el1, er1, g1 = _mm_att(x, W1, al1.reshape(1, -1), ar1.reshape(1, -1))
    numc1, sc1 = _sc_layer(src, dst, el1, er1, g1, h1)

    h2, el2, er2, g2 = _norm_mm_att(numc1, sc1, b1.reshape(1, -1), W2,
                                    al2.reshape(1, -1), ar2.reshape(1, -1))
    numc2, sc2 = _sc_layer(src, dst, el2, er2, g2, h2)

    (out,) = _head(numc2, sc2, b2.reshape(1, -1), fc1_w, fc1_b.reshape(1, -1),
                   fc2_w, fc2_b.reshape(1, -1))
    return out.T


# speedup vs baseline: 7.5213x; 1.0741x over previous
import functools

import jax
import jax.numpy as jnp
from jax import lax
from jax.experimental import pallas as pl
from jax.experimental.pallas import tpu as pltpu
from jax.experimental.pallas import tpu_sc as plsc

_N = 10000
_E = 160000
_D = 256
_NC = 2
_NS = 16
_L = 16
_F = _D // 4
_ACC = 10240
_RPT = _ACC // _NS
_FPT = _RPT * _F
_SV = 8192
_K = 80
_SUB = 2000
_EPT = _E // _NS
_NCH = _EPT // _K
_BLK = 1000
_NBLK = _N // _BLK

def _iota16():
    return lax.iota(jnp.int32, _L)


def _splat(v16, j):
    idx = (_iota16() * 0 + j).reshape(_L, 1)
    return lax.gather(
        v16, idx,
        lax.GatherDimensionNumbers(offset_dims=(), collapsed_slice_dims=(0,),
                                   start_index_map=(0,)),
        (1,), mode=lax.GatherScatterMode.PROMISE_IN_BOUNDS)


def _sc_edge_body(src_hbm, dst_hbm, el_hbm, er_hbm, g_hbm, h_hbm,
                  num_out, s_out,
                  srcv, dstv, giv, av, bv, rows, vals, offs, gv, sv, sem,
                  semr, sems, semi, sema, acc, sacc):
    cid = lax.axis_index("c")
    sid = lax.axis_index("s")

    def zb(i, cc):
        sv[pl.ds(i * _L, _L)] = lax.convert_element_type(_iota16() * 0,
                                                         jnp.float32)
        return cc

    lax.fori_loop(0, _SV // _L, zb, 0)

    def zcp(i, cc):
        pltpu.sync_copy(sv, acc.at[pl.ds(sid * _FPT + i * _SV, _SV)])
        return cc

    lax.fori_loop(0, _FPT // _SV, zcp, 0)
    pltpu.sync_copy(sv.at[pl.ds(0, _RPT)], sacc.at[pl.ds(sid * _RPT, _RPT)])
    pltpu.sync_copy(g_hbm, gv)
    plsc.subcore_barrier()

    tile_e0 = sid * _EPT
    nsub = _EPT // _SUB

    def sub0(sb):
        return jnp.minimum(tile_e0 + sb * _SUB, _E - _SUB)

    def idx_cps(sb, p):
        s0 = sub0(sb)
        ps = pl.ds(p * _SUB, _SUB)
        return (pltpu.make_async_copy(src_hbm.at[pl.ds(s0, _SUB)],
                                      srcv.at[ps], semi),
                pltpu.make_async_copy(dst_hbm.at[pl.ds(s0, _SUB)],
                                      dstv.at[ps], semi))

    def ab_cps(p):
        ps = pl.ds(p * _SUB, _SUB)
        return (pltpu.make_async_copy(el_hbm.at[srcv.at[ps]], av.at[ps], sema),
                pltpu.make_async_copy(er_hbm.at[dstv.at[ps]], bv.at[ps], sema))

    pltpu.sync_copy(src_hbm.at[pl.ds(tile_e0, _SUB)], srcv.at[pl.ds(0, _SUB)])
    pltpu.sync_copy(dst_hbm.at[pl.ds(tile_e0, _SUB)], dstv.at[pl.ds(0, _SUB)])
    for cp in ab_cps(0):
        cp.start()

    def sub_body(sb, carry0):
        p = lax.rem(sb, 2)
        pb = p * _SUB
        for cp in idx_cps(sb + 1, 1 - p):
            cp.start()
        for cp in ab_cps(p):
            cp.wait()

        def grp(gi, cc):
            s16 = pl.ds(pb + gi * _L, _L)
            a = av[s16]
            b = bv[s16]
            e = a + b
            e = jnp.maximum(e, 0.2 * e)
            m = gv[...] + b
            m = jnp.maximum(m, 0.2 * m)
            av[s16] = jnp.exp(e - m)
            giv[pl.ds(gi * _L, _L)] = srcv[s16] + cid * _N
            return cc

        lax.fori_loop(0, _SUB // _L, grp, 0)
        pltpu.sync_copy(av.at[pl.ds(pb, _SUB)],
                        sacc.at[dstv.at[pl.ds(pb, _SUB)]], add=True)

        kfsz = _K * _F
        nchs = _SUB // _K

        def rows_cp(c, b):
            return pltpu.make_async_copy(
                h_hbm.at[giv.at[pl.ds(c * _K, _K)]],
                rows.at[pl.ds(b * _K, _K)], semr)

        def scat_cp(b):
            return pltpu.make_async_copy(
                vals.at[pl.ds(b * kfsz, kfsz)],
                acc.at[offs.at[pl.ds(b * kfsz, kfsz)]], sems)

        pltpu.async_copy(h_hbm.at[giv.at[pl.ds(0, _K)]],
                         rows.at[pl.ds(0, _K)], semr)

        def chunk_body(c, carry):
            b = lax.rem(c, 2)
            rows_cp(c, b).wait()

            @pl.when(c == 0)
            def _():
                for cp in idx_cps(sb + 1, 1 - p):
                    cp.wait()
                for cp in ab_cps(1 - p):
                    cp.start()

            @pl.when(c + 1 < nchs)
            def _():
                pltpu.async_copy(
                    h_hbm.at[giv.at[pl.ds((c + 1) * _K, _K)]],
                    rows.at[pl.ds((1 - b) * _K, _K)], semr)

            @pl.when(c >= 2)
            def _():
                scat_cp(b).wait()

            def scale(gi, cc):
                base16 = pl.ds(pb + c * _K + gi * _L, _L)
                ex16 = av[base16]
                dof16 = dstv[base16] * _F
                for j in range(_L):
                    w = _splat(ex16, j)
                    dof = _splat(dof16, j)
                    k = gi * _L + j
                    for q in range(_F // _L):
                        kf = pl.ds(b * kfsz + k * _F + q * _L, _L)
                        vals[kf] = rows[b * _K + k, pl.ds(q * _L, _L)] * w
                        offs[kf] = dof + (_iota16() + q * _L)
                return cc

            lax.fori_loop(0, _K // _L, scale, 0)
            pltpu.async_copy(vals.at[pl.ds(b * kfsz, kfsz)],
                             acc.at[offs.at[pl.ds(b * kfsz, kfsz)]], sems,
                             add=True)
            return carry

        lax.fori_loop(0, nchs, chunk_body, 0)
        scat_cp((nchs - 2) % 2).wait()
        scat_cp((nchs - 1) % 2).wait()
        return carry0

    lax.fori_loop(0, nsub, sub_body, 0)
    pf = lax.rem(jnp.int32(nsub), 2)
    for cp in ab_cps(pf):
        cp.wait()
    plsc.subcore_barrier()

    def ocp(i, cc):
        pltpu.sync_copy(acc.at[pl.ds(sid * _FPT + i * _SV, _SV)], sv)
        pltpu.sync_copy(sv, num_out.at[
            pl.ds(cid * _ACC * _F + sid * _FPT + i * _SV, _SV)])
        return cc

    lax.fori_loop(0, _FPT // _SV, ocp, 0)
    pltpu.sync_copy(sacc.at[pl.ds(sid * _RPT, _RPT)], sv.at[pl.ds(0, _RPT)])
    pltpu.sync_copy(sv.at[pl.ds(0, _RPT)],
                    s_out.at[pl.ds(cid * _ACC + sid * _RPT, _RPT)])


@functools.cache
def _get_sc_edge():
    return functools.partial(
        pl.kernel,
        out_type=(
            jax.ShapeDtypeStruct((_NC * _ACC * _F,), jnp.float32),
            jax.ShapeDtypeStruct((_NC * _ACC,), jnp.float32),
        ),
        mesh=plsc.VectorSubcoreMesh(core_axis_name="c", subcore_axis_name="s",
                                    num_cores=_NC, num_subcores=_NS),
        compiler_params=pltpu.CompilerParams(use_tc_tiling_on_sc=False),
        scratch_types=[
            pltpu.VMEM((2 * _SUB,), jnp.int32),
            pltpu.VMEM((2 * _SUB,), jnp.int32),
            pltpu.VMEM((_SUB,), jnp.int32),
            pltpu.VMEM((2 * _SUB,), jnp.float32),
            pltpu.VMEM((2 * _SUB,), jnp.float32),
            pltpu.VMEM((2 * _K, _F), jnp.float32),
            pltpu.VMEM((2 * _K * _F,), jnp.float32),
            pltpu.VMEM((2 * _K * _F,), jnp.int32),
            pltpu.VMEM((_L,), jnp.float32),
            pltpu.VMEM((_SV,), jnp.float32),
            pltpu.SemaphoreType.DMA,
            pltpu.SemaphoreType.DMA,
            pltpu.SemaphoreType.DMA,
            pltpu.SemaphoreType.DMA,
            pltpu.SemaphoreType.DMA,
            pltpu.VMEM_SHARED((_ACC * _F,), jnp.float32),
            pltpu.VMEM_SHARED((_ACC,), jnp.float32),
        ],
    )(_sc_edge_body)


_SELU_L = 1.0507009873554805
_SELU_A = 1.6732632423543772


def _selu(x):
    return _SELU_L * jnp.where(x > 0, x, _SELU_A * (jnp.exp(jnp.minimum(x, 0.0)) - 1.0))


def _mm_att_body(x_ref, w_ref, al_ref, ar_ref, h_ref, el_ref, er_ref, g_ref, mx_ref):
    i = pl.program_id(0)
    h = jnp.dot(x_ref[...], w_ref[...], preferred_element_type=jnp.float32,
                precision=lax.Precision.HIGHEST)
    h_ref[...] = h
    el = jnp.sum(h * al_ref[...], axis=1, keepdims=True)
    er = jnp.sum(h * ar_ref[...], axis=1, keepdims=True)
    el_ref[...] = el
    er_ref[...] = er
    bm = jnp.max(el)

    @pl.when(i == 0)
    def _():
        mx_ref[0] = bm

    @pl.when(i > 0)
    def _():
        mx_ref[0] = jnp.maximum(mx_ref[0], bm)

    @pl.when(i == pl.num_programs(0) - 1)
    def _():
        g_ref[...] = jnp.full((1, 16), mx_ref[0], jnp.float32)


def _norm_mm_att_body(num_ref, s_ref, b_ref, w_ref, al_ref, ar_ref,
                      h_ref, el_ref, er_ref, g_ref, mx_ref):
    i = pl.program_id(0)
    xin = _selu(num_ref[...] / (s_ref[...] + 1e-16) + b_ref[...])
    h = jnp.dot(xin, w_ref[...], preferred_element_type=jnp.float32,
                precision=lax.Precision.HIGHEST)
    h_ref[...] = h
    el = jnp.sum(h * al_ref[...], axis=1, keepdims=True)
    er = jnp.sum(h * ar_ref[...], axis=1, keepdims=True)
    el_ref[...] = el
    er_ref[...] = er
    bm = jnp.max(el)

    @pl.when(i == 0)
    def _():
        mx_ref[0] = bm

    @pl.when(i > 0)
    def _():
        mx_ref[0] = jnp.maximum(mx_ref[0], bm)

    @pl.when(i == pl.num_programs(0) - 1)
    def _():
        g_ref[...] = jnp.full((1, 16), mx_ref[0], jnp.float32)


def _head_body(num_ref, s_ref, b_ref, f1w_ref, f1b_ref, f2w_ref, f2b_ref,
               out_ref, acc_ref):
    i = pl.program_id(0)
    xin = _selu(num_ref[...] / (s_ref[...] + 1e-16) + b_ref[...])
    part = jnp.sum(xin, axis=0, keepdims=True)

    @pl.when(i == 0)
    def _():
        acc_ref[...] = part

    @pl.when(i > 0)
    def _():
        acc_ref[...] = acc_ref[...] + part

    @pl.when(i == pl.num_programs(0) - 1)
    def _():
        gbar = acc_ref[...] / _N
        z1 = _selu(jnp.dot(gbar, f1w_ref[...], preferred_element_type=jnp.float32,
                           precision=lax.Precision.HIGHEST) + f1b_ref[...])
        z2 = jnp.dot(z1, f2w_ref[...], preferred_element_type=jnp.float32,
                     precision=lax.Precision.HIGHEST) + f2b_ref[...]
        zm = jnp.max(z2)
        out_ref[...] = (z2 - zm) - jnp.log(jnp.sum(jnp.exp(z2 - zm)))


def _row_spec(width):
    return pl.BlockSpec((_BLK, width), lambda i: (i, 0))


def _full_spec(shape):
    return pl.BlockSpec(shape, lambda i: tuple(0 for _ in shape))


_mm_att = pl.pallas_call(
    _mm_att_body,
    grid=(_NBLK,),
    in_specs=[
        _row_spec(_D),
        _full_spec((_D, _D)),
        _full_spec((1, _D)),
        _full_spec((1, _D)),
    ],
    out_specs=[
        _row_spec(_D),
        _row_spec(1),
        _row_spec(1),
        _full_spec((1, 16)),
    ],
    out_shape=[
        jax.ShapeDtypeStruct((_N, _D), jnp.float32),
        jax.ShapeDtypeStruct((_N, 1), jnp.float32),
        jax.ShapeDtypeStruct((_N, 1), jnp.float32),
        jax.ShapeDtypeStruct((1, 16), jnp.float32),
    ],
    scratch_shapes=[pltpu.SMEM((1,), jnp.float32)],
)

_norm_mm_att = pl.pallas_call(
    _norm_mm_att_body,
    grid=(_NBLK,),
    in_specs=[
        _row_spec(_D),
        _row_spec(1),
        _full_spec((1, _D)),
        _full_spec((_D, _D)),
        _full_spec((1, _D)),
        _full_spec((1, _D)),
    ],
    out_specs=[
        _row_spec(_D),
        _row_spec(1),
        _row_spec(1),
        _full_spec((1, 16)),
    ],
    out_shape=[
        jax.ShapeDtypeStruct((_N, _D), jnp.float32),
        jax.ShapeDtypeStruct((_N, 1), jnp.float32),
        jax.ShapeDtypeStruct((_N, 1), jnp.float32),
        jax.ShapeDtypeStruct((1, 16), jnp.float32),
    ],
    scratch_shapes=[pltpu.SMEM((1,), jnp.float32)],
)

_head = pl.pallas_call(
    _head_body,
    grid=(_NBLK,),
    in_specs=[
        _row_spec(_D),
        _row_spec(1),
        _full_spec((1, _D)),
        _full_spec((_D, _D // 2)),
        _full_spec((1, _D // 2)),
        _full_spec((_D // 2, 2)),
        _full_spec((1, 2)),
    ],
    out_specs=[_full_spec((1, 2))],
    out_shape=[jax.ShapeDtypeStruct((1, 2), jnp.float32)],
    scratch_shapes=[pltpu.VMEM((1, _D), jnp.float32)],
)


def _split_tables(h):
    ta = jnp.concatenate([h[:, :_F], h[:, _F:2 * _F]], axis=0)
    tb = jnp.concatenate([h[:, 2 * _F:3 * _F], h[:, 3 * _F:]], axis=0)
    return ta, tb


def _sc_layer(src, dst, el, er, g, h):
    ta, tb = _split_tables(h)
    sc_edge = _get_sc_edge()
    args = (src, dst, el.reshape(-1), er.reshape(-1), g.reshape(-1))
    na, s = sc_edge(*args, ta)
    nb, _ = sc_edge(*args, tb)
    na = na.reshape(_NC, _ACC, _F)
    nb = nb.reshape(_NC, _ACC, _F)
    numc = jnp.concatenate(
        [na[0, :_N], na[1, :_N], nb[0, :_N], nb[1, :_N]], axis=1)
    return numc, s[:_N].reshape(-1, 1)


def kernel(x, edge_index, W1, al1, ar1, b1, W2, al2, ar2, b2,
           fc1_w, fc1_b, fc2_w, fc2_b):
    src = edge_index[0]
    dst = edge_index[1]

    h1, el1, er1, g1 = _mm_att(x, W1, al1.reshape(1, -1), ar1.reshape(1, -1))
    numc1, sc1 = _sc_layer(src, dst, el1, er1, g1, h1)

    h2, el2, er2, g2 = _norm_mm_att(numc1, sc1, b1.reshape(1, -1), W2,
                                    al2.reshape(1, -1), ar2.reshape(1, -1))
    numc2, sc2 = _sc_layer(src, dst, el2, er2, g2, h2)

    (out,) = _head(numc2, sc2, b2.reshape(1, -1), fc1_w, fc1_b.reshape(1, -1),
                   fc2_w, fc2_b.reshape(1, -1))
    return out.T
